# baseline (speedup 1.0000x reference)
import functools

import jax
import jax.numpy as jnp
from jax import lax
from jax.experimental import pallas as pl
from jax.experimental.pallas import tpu as pltpu
from jax.experimental.pallas import tpu_sc as plsc

N = 10000
E = 320000
H = 256
CW = 80
ERP = 4096
RCH = 8
NSPAD = 10240
PADIDX = 10200
NPAD = 10112
RPT = NPAD // 16
ZR = 8

_f32 = jnp.float32


def _mesh():
    return plsc.VectorSubcoreMesh(core_axis_name="c", subcore_axis_name="s")


def _sc_degrees(srcA, dstA):
    rows_per_tec = ERP // 32
    n_chunks = rows_per_tec // RCH

    @functools.partial(
        pl.kernel,
        out_type=jax.ShapeDtypeStruct((4 * NSPAD,), _f32),
        mesh=_mesh(),
        compiler_params=pltpu.CompilerParams(needs_layout_passes=False),
        scratch_types=[
            pltpu.VMEM((RCH, CW), jnp.int32),
            pltpu.VMEM((RCH, CW), jnp.int32),
            pltpu.VMEM((CW,), _f32),
            pltpu.VMEM((NSPAD // 16,), _f32),
            pltpu.VMEM_SHARED((NSPAD,), _f32),
            pltpu.VMEM_SHARED((NSPAD,), _f32),
        ],
    )
    def k(s_h, d_h, out_h, sidx_v, didx_v, ones_v, z_v, sdeg_sh, ddeg_sh):
        cid = lax.axis_index("c")
        sid = lax.axis_index("s")

        def fill_ones(i, _):
            ones_v[pl.ds(i * 16, 16)] = jnp.ones((16,), _f32)
            return 0

        lax.fori_loop(0, CW // 16, fill_ones, 0)

        def fill_z(i, _):
            z_v[pl.ds(i * 16, 16)] = jnp.zeros((16,), _f32)
            return 0

        lax.fori_loop(0, NSPAD // 16 // 16, fill_z, 0)

        seg = NSPAD // 16
        pltpu.sync_copy(z_v, sdeg_sh.at[pl.ds(sid * seg, seg)])
        pltpu.sync_copy(z_v, ddeg_sh.at[pl.ds(sid * seg, seg)])
        plsc.subcore_barrier()

        base = cid * (ERP // 2) + sid * rows_per_tec

        def chunk(t, _):
            r0 = base + t * RCH
            pltpu.sync_copy(s_h.at[pl.ds(r0, RCH)], sidx_v)
            pltpu.sync_copy(d_h.at[pl.ds(r0, RCH)], didx_v)
            for j in range(RCH):
                pltpu.sync_copy(ones_v, sdeg_sh.at[sidx_v.at[j]], add=True)
                pltpu.sync_copy(ones_v, ddeg_sh.at[didx_v.at[j]], add=True)
            return 0

        lax.fori_loop(0, n_chunks, chunk, 0)
        plsc.subcore_barrier()

        pltpu.sync_copy(sdeg_sh.at[pl.ds(sid * seg, seg)],
                        out_h.at[pl.ds(cid * NSPAD + sid * seg, seg)])
        pltpu.sync_copy(ddeg_sh.at[pl.ds(sid * seg, seg)],
                        out_h.at[pl.ds(2 * NSPAD + cid * NSPAD + sid * seg, seg)])

    return k(srcA, dstA)


def _sc_coeff(srcB, dstB, ew2d, norms_flat):
    rows_per_tec = ERP // 32
    n_chunks = rows_per_tec // RCH

    @functools.partial(
        pl.kernel,
        out_type=jax.ShapeDtypeStruct((ERP, CW), _f32),
        mesh=_mesh(),
        compiler_params=pltpu.CompilerParams(needs_layout_passes=False),
        scratch_types=[
            pltpu.VMEM((RCH, CW), jnp.int32),
            pltpu.VMEM((RCH, CW), jnp.int32),
            pltpu.VMEM((RCH, CW), _f32),
            pltpu.VMEM((RCH, CW), _f32),
            pltpu.VMEM((NSPAD,), _f32),
            pltpu.VMEM((NSPAD,), _f32),
        ],
    )
    def k(s_h, d_h, ew_h, norms_h, c_out_h, sidx_v, didx_v, ew_v, c_v, ns_v, nd_v):
        cid = lax.axis_index("c")
        sid = lax.axis_index("s")
        pltpu.sync_copy(norms_h.at[pl.ds(0, NSPAD)], ns_v)
        pltpu.sync_copy(norms_h.at[pl.ds(NSPAD, NSPAD)], nd_v)

        base = (cid * 16 + sid) * rows_per_tec

        def chunk(t, _):
            r0 = base + t * RCH
            pltpu.sync_copy(s_h.at[pl.ds(r0, RCH)], sidx_v)
            pltpu.sync_copy(d_h.at[pl.ds(r0, RCH)], didx_v)
            pltpu.sync_copy(ew_h.at[pl.ds(r0, RCH)], ew_v)
            for j in range(RCH):
                def g16(g, _):
                    si = sidx_v[j, pl.ds(g * 16, 16)]
                    di = didx_v[j, pl.ds(g * 16, 16)]
                    a = plsc.load_gather(ns_v, [si])
                    b = plsc.load_gather(nd_v, [di])
                    c_v[j, pl.ds(g * 16, 16)] = ew_v[j, pl.ds(g * 16, 16)] * a * b
                    return 0

                lax.fori_loop(0, CW // 16, g16, 0)
            pltpu.sync_copy(c_v, c_out_h.at[pl.ds(r0, RCH)])
            return 0

        lax.fori_loop(0, n_chunks, chunk, 0)

    return k(srcB, dstB, ew2d, norms_flat)


def _sc_spmm(table, srcB, dstB, c2d, *, split_edges):
    D = 128
    rows_per_tec = (ERP // 32) if split_edges else (ERP // 16)
    n_chunks = rows_per_tec // RCH

    @functools.partial(
        pl.kernel,
        out_type=jax.ShapeDtypeStruct((2, NPAD, D), _f32),
        mesh=_mesh(),
        compiler_params=pltpu.CompilerParams(needs_layout_passes=False),
        scratch_types=[
            pltpu.VMEM((2, RCH, CW), jnp.int32),
            pltpu.VMEM((2, RCH, CW), jnp.int32),
            pltpu.VMEM((2 * RCH * CW,), _f32),
            pltpu.VMEM((4, CW, D), _f32),
            pltpu.VMEM((ZR, D), _f32),
            pltpu.VMEM_SHARED((NPAD, D), _f32),
            pltpu.SemaphoreType.DMA,
            pltpu.SemaphoreType.DMA,
            pltpu.SemaphoreType.DMA,
            pltpu.SemaphoreType.DMA,
            pltpu.SemaphoreType.DMA,
            pltpu.SemaphoreType.DMA,
            pltpu.SemaphoreType.DMA,
            pltpu.SemaphoreType.DMA,
            pltpu.SemaphoreType.DMA,
        ],
    )
    def k(table_h, s_h, d_h, c_h, out_h, sidx_v, didx_v, c_v, rows_v, z_v,
          agg_sh, gs0, gs1, gs2, gs3, ss0, ss1, ss2, ss3, isem):
        gsems = (gs0, gs1, gs2, gs3)
        ssems = (ss0, ss1, ss2, ss3)
        cid = lax.axis_index("c")
        sid = lax.axis_index("s")

        def zb(r, _):
            for g in range(D // 16):
                z_v[r, pl.ds(g * 16, 16)] = jnp.zeros((16,), _f32)
            return 0

        lax.fori_loop(0, ZR, zb, 0)

        def zc(t, _):
            pltpu.sync_copy(z_v, agg_sh.at[pl.ds(sid * RPT + t * ZR, ZR)])
            return 0

        lax.fori_loop(0, RPT // ZR, zc, 0)
        plsc.subcore_barrier()

        if split_edges:
            base = cid * (ERP // 2) + sid * rows_per_tec
        else:
            base = sid * rows_per_tec
        off = cid * NPAD

        def idx_descs(slot, r0):
            return (
                pltpu.make_async_copy(s_h.at[pl.ds(r0, RCH)], sidx_v.at[slot], isem),
                pltpu.make_async_copy(d_h.at[pl.ds(r0, RCH)], didx_v.at[slot], isem),
                pltpu.make_async_copy(c_h.at[pl.ds(r0 * CW, RCH * CW)],
                                      c_v.at[pl.ds(slot * RCH * CW, RCH * CW)],
                                      isem),
            )

        def addoff(slot):
            def body(j, _):
                for g in range(CW // 16):
                    sidx_v[slot, j, pl.ds(g * 16, 16)] = (
                        sidx_v[slot, j, pl.ds(g * 16, 16)] + off)
                return 0

            lax.fori_loop(0, RCH, body, 0)

        def fire_gather(slot, j, b):
            pltpu.make_async_copy(
                table_h.at[sidx_v.at[slot, j]], rows_v.at[b], gsems[b]).start()

        def gather_wait(b):
            pltpu.make_async_copy(
                table_h.at[sidx_v.at[0, 0]], rows_v.at[b], gsems[b]).wait()

        def scatter_wait(b):
            pltpu.make_async_copy(
                rows_v.at[b], agg_sh.at[didx_v.at[0, 0]], ssems[b]).wait()

        for d in idx_descs(0, base):
            d.start()
        for d in idx_descs(0, base):
            d.wait()
        addoff(0)
        fire_gather(0, 0, 0)
        fire_gather(0, 1, 1)

        def chunk(t, _):
            slot = lax.rem(t, 2)
            for j in range(RCH):
                b = j % 4
                b2 = (j + 2) % 4
                if j >= 2:
                    scatter_wait(b2)
                else:
                    @pl.when(t > 0)
                    def _():
                        scatter_wait(b2)

                if j == 2:
                    @pl.when(t + 1 < n_chunks)
                    def _():
                        for d in idx_descs(1 - slot, base + (t + 1) * RCH):
                            d.start()

                if j < RCH - 2:
                    fire_gather(slot, j + 2, b2)
                elif j == RCH - 2:
                    @pl.when(t + 1 < n_chunks)
                    def _():
                        for d in idx_descs(1 - slot, base + (t + 1) * RCH):
                            d.wait()
                        addoff(1 - slot)
                        fire_gather(1 - slot, 0, b2)
                else:
                    @pl.when(t + 1 < n_chunks)
                    def _():
                        fire_gather(1 - slot, 1, b2)

                gather_wait(b)
                cbase = (slot * RCH + j) * CW

                def scale(i4, _):
                    for u in range(4):
                        i = i4 * 4 + u
                        ci = jnp.full((16,), cbase + i, jnp.int32)
                        cs = plsc.load_gather(c_v, [ci])
                        for g_ in range(D // 16):
                            rows_v[b, i, pl.ds(g_ * 16, 16)] = (
                                rows_v[b, i, pl.ds(g_ * 16, 16)] * cs)
                    return 0

                lax.fori_loop(0, CW // 4, scale, 0)
                pltpu.make_async_copy(
                    rows_v.at[b], agg_sh.at[didx_v.at[slot, j]],
                    ssems[b]).start(add=True)
            return 0

        lax.fori_loop(0, n_chunks, chunk, 0)
        scatter_wait(2)
        scatter_wait(3)
        plsc.subcore_barrier()

        def co(t, _):
            r0 = sid * RPT + t * ZR
            pltpu.sync_copy(agg_sh.at[pl.ds(r0, ZR)], out_h.at[cid, pl.ds(r0, ZR)])
            return 0

        lax.fori_loop(0, RPT // ZR, co, 0)

    return k(table, srcB, dstB, c2d)


def _sc_pred(srcB, dstB, p12):
    rows_per_tec = ERP // 32
    n_chunks = rows_per_tec // RCH

    @functools.partial(
        pl.kernel,
        out_type=jax.ShapeDtypeStruct((ERP, CW), _f32),
        mesh=_mesh(),
        compiler_params=pltpu.CompilerParams(needs_layout_passes=False),
        scratch_types=[
            pltpu.VMEM((RCH, CW), jnp.int32),
            pltpu.VMEM((RCH, CW), jnp.int32),
            pltpu.VMEM((RCH, CW), _f32),
            pltpu.VMEM((N,), _f32),
            pltpu.VMEM((N,), _f32),
        ],
    )
    def k(s_h, d_h, p_h, out_h, sidx_v, didx_v, sc_v, p1_v, p2_v):
        cid = lax.axis_index("c")
        sid = lax.axis_index("s")
        pltpu.sync_copy(p_h.at[pl.ds(0, N)], p1_v)
        pltpu.sync_copy(p_h.at[pl.ds(N, N)], p2_v)

        base = (cid * 16 + sid) * rows_per_tec

        def chunk(t, _):
            r0 = base + t * RCH
            pltpu.sync_copy(s_h.at[pl.ds(r0, RCH)], sidx_v)
            pltpu.sync_copy(d_h.at[pl.ds(r0, RCH)], didx_v)
            for j in range(RCH):
                def g16(g, _):
                    si = sidx_v[j, pl.ds(g * 16, 16)]
                    di = didx_v[j, pl.ds(g * 16, 16)]
                    a = plsc.load_gather(p1_v, [si])
                    b = plsc.load_gather(p2_v, [di])
                    sc_v[j, pl.ds(g * 16, 16)] = a + b
                    return 0

                lax.fori_loop(0, CW // 16, g16, 0)
            pltpu.sync_copy(sc_v, out_h.at[pl.ds(r0, RCH)])
            return 0

        lax.fori_loop(0, n_chunks, chunk, 0)

    return k(srcB, dstB, p12)


def _tc_ew(e, w1, b1r, w2, b2r):
    BE = 8000

    def body(e_ref, w1_ref, b1_ref, w2_ref, b2_ref, out_ref):
        wc = jnp.dot(w1_ref[...], w2_ref[...], preferred_element_type=_f32)
        bc = jnp.dot(b1_ref[...], w2_ref[...], preferred_element_type=_f32)
        out_ref[...] = (
            jnp.dot(e_ref[...], wc, preferred_element_type=_f32)
            + bc + b2_ref[...])

    return pl.pallas_call(
        body,
        grid=(E // BE,),
        in_specs=[
            pl.BlockSpec((BE, 16), lambda i: (i, 0)),
            pl.BlockSpec((16, 8), lambda i: (0, 0)),
            pl.BlockSpec((1, 8), lambda i: (0, 0)),
            pl.BlockSpec((8, 1), lambda i: (0, 0)),
            pl.BlockSpec((1, 1), lambda i: (0, 0)),
        ],
        out_specs=pl.BlockSpec((BE, 1), lambda i: (i, 0)),
        out_shape=jax.ShapeDtypeStruct((E, 1), _f32),
    )(e, w1, b1r, w2, b2r)


def _tc_norms(degs):
    def body(d_ref, out_ref):
        s = jnp.maximum(d_ref[0, :] + d_ref[1, :], 1.0)
        d = jnp.maximum(d_ref[2, :] + d_ref[3, :], 1.0)
        out_ref[0, :] = lax.rsqrt(s)
        out_ref[1, :] = lax.rsqrt(d)

    return pl.pallas_call(
        body,
        out_shape=jax.ShapeDtypeStruct((2, NSPAD), _f32),
    )(degs.reshape(4, NSPAD))


def _tc_mm1(agg1, w1, b1r):
    BN = RPT

    def body(a_ref, w_ref, b_ref, out_ref):
        acc = a_ref[0] + a_ref[1]
        out_ref[...] = jax.nn.relu(
            jnp.dot(acc, w_ref[...], preferred_element_type=_f32) + b_ref[...])

    nb = NPAD // BN
    return pl.pallas_call(
        body,
        grid=(nb, 2),
        in_specs=[
            pl.BlockSpec((2, BN, 128), lambda i, j: (0, i, 0)),
            pl.BlockSpec((128, 128), lambda i, j: (0, j)),
            pl.BlockSpec((1, 128), lambda i, j: (0, j)),
        ],
        out_specs=pl.BlockSpec((BN, 128), lambda i, j: (j * nb + i, 0)),
        out_shape=jax.ShapeDtypeStruct((2 * NPAD, 128), _f32),
    )(agg1, w1, b1r)


def _tc_mm2(agg2, w2t, w2b, b2r, wcat, bcat):
    BN = RPT

    def body(a_ref, wt_ref, wb_ref, b_ref, wc_ref, bc_ref, out_ref):
        h2 = jax.nn.relu(
            jnp.dot(a_ref[0], wt_ref[...], preferred_element_type=_f32)
            + jnp.dot(a_ref[1], wb_ref[...], preferred_element_type=_f32)
            + b_ref[...])
        out_ref[...] = (
            jnp.dot(h2, wc_ref[...], preferred_element_type=_f32) + bc_ref[...])

    nb = NPAD // BN
    return pl.pallas_call(
        body,
        grid=(nb,),
        in_specs=[
            pl.BlockSpec((2, BN, 128), lambda i: (0, i, 0)),
            pl.BlockSpec((128, H), lambda i: (0, 0)),
            pl.BlockSpec((128, H), lambda i: (0, 0)),
            pl.BlockSpec((1, H), lambda i: (0, 0)),
            pl.BlockSpec((H, 128), lambda i: (0, 0)),
            pl.BlockSpec((1, 128), lambda i: (0, 0)),
        ],
        out_specs=pl.BlockSpec((BN, 128), lambda i: (i, 0)),
        out_shape=jax.ShapeDtypeStruct((NPAD, 128), _f32),
    )(agg2, w2t, w2b, b2r, wcat, bcat)


def _pad2d(v, fill):
    pad = jnp.full((ERP * CW - E,), fill, v.dtype)
    return jnp.concatenate([v, pad]).reshape(ERP, CW)


def kernel(x, edge_index, e, W_lin1, b_lin1, W_lin2, b_lin2,
           W_conv1, b_conv1, W_conv2, b_conv2, W_pred, b_pred):
    src = edge_index[0]
    dst = edge_index[1]
    srcA = _pad2d(src, PADIDX)
    dstA = _pad2d(dst, PADIDX)
    srcB = _pad2d(src, 0)
    dstB = _pad2d(dst, 0)

    degs = _sc_degrees(srcA, dstA)
    norms = _tc_norms(degs).reshape(2 * NSPAD)

    ew = _tc_ew(e, W_lin1, b_lin1.reshape(1, 8), W_lin2, b_lin2.reshape(1, 1))
    c2d = _sc_coeff(srcB, dstB, _pad2d(ew.reshape(E), 0.0), norms)

    xp = jnp.concatenate([x, jnp.zeros((NPAD - N, 128), _f32)])
    xp2 = jnp.concatenate([xp, xp])
    cflat = c2d.reshape(ERP * CW)
    agg1 = _sc_spmm(xp2, srcB, dstB, cflat, split_edges=True)
    h1cat = _tc_mm1(agg1, W_conv1, b_conv1.reshape(1, H))
    agg2 = _sc_spmm(h1cat, srcB, dstB, cflat, split_edges=False)

    wcat = jnp.zeros((H, 128), _f32)
    wcat = wcat.at[:, 0].set(W_pred[:H, 0]).at[:, 1].set(W_pred[H:, 0])
    bcat = jnp.zeros((1, 128), _f32).at[0, 0].set(b_pred[0])
    P = _tc_mm2(agg2, W_conv2[:128], W_conv2[128:], b_conv2.reshape(1, H),
                wcat, bcat)

    p12 = jnp.concatenate([P[:N, 0], P[:N, 1]])
    score2d = _sc_pred(srcB, dstB, p12)
    return score2d.reshape(ERP * CW)[:E].reshape(E, 1)


# speedup vs baseline: 1.0366x; 1.0366x over previous
import functools

import jax
import jax.numpy as jnp
from jax import lax
from jax.experimental import pallas as pl
from jax.experimental.pallas import tpu as pltpu
from jax.experimental.pallas import tpu_sc as plsc

N = 10000
E = 320000
H = 256
CW = 80
ERP = 4096
RCH = 8
NSPAD = 10240
PADIDX = 10200
NPAD = 10112
RPT = NPAD // 16
ZR = 8

_f32 = jnp.float32


def _mesh():
    return plsc.VectorSubcoreMesh(core_axis_name="c", subcore_axis_name="s")


def _sc_degrees(srcA, dstA):
    rows_per_tec = ERP // 32
    n_chunks = rows_per_tec // RCH

    @functools.partial(
        pl.kernel,
        out_type=jax.ShapeDtypeStruct((4 * NSPAD,), _f32),
        mesh=_mesh(),
        compiler_params=pltpu.CompilerParams(needs_layout_passes=False),
        scratch_types=[
            pltpu.VMEM((RCH, CW), jnp.int32),
            pltpu.VMEM((RCH, CW), jnp.int32),
            pltpu.VMEM((CW,), _f32),
            pltpu.VMEM((NSPAD // 16,), _f32),
            pltpu.VMEM_SHARED((NSPAD,), _f32),
            pltpu.VMEM_SHARED((NSPAD,), _f32),
        ],
    )
    def k(s_h, d_h, out_h, sidx_v, didx_v, ones_v, z_v, sdeg_sh, ddeg_sh):
        cid = lax.axis_index("c")
        sid = lax.axis_index("s")

        def fill_ones(i, _):
            ones_v[pl.ds(i * 16, 16)] = jnp.ones((16,), _f32)
            return 0

        lax.fori_loop(0, CW // 16, fill_ones, 0)

        def fill_z(i, _):
            z_v[pl.ds(i * 16, 16)] = jnp.zeros((16,), _f32)
            return 0

        lax.fori_loop(0, NSPAD // 16 // 16, fill_z, 0)

        seg = NSPAD // 16
        pltpu.sync_copy(z_v, sdeg_sh.at[pl.ds(sid * seg, seg)])
        pltpu.sync_copy(z_v, ddeg_sh.at[pl.ds(sid * seg, seg)])
        plsc.subcore_barrier()

        base = cid * (ERP // 2) + sid * rows_per_tec

        def chunk(t, _):
            r0 = base + t * RCH
            pltpu.sync_copy(s_h.at[pl.ds(r0, RCH)], sidx_v)
            pltpu.sync_copy(d_h.at[pl.ds(r0, RCH)], didx_v)
            for j in range(RCH):
                pltpu.sync_copy(ones_v, sdeg_sh.at[sidx_v.at[j]], add=True)
                pltpu.sync_copy(ones_v, ddeg_sh.at[didx_v.at[j]], add=True)
            return 0

        lax.fori_loop(0, n_chunks, chunk, 0)
        plsc.subcore_barrier()

        pltpu.sync_copy(sdeg_sh.at[pl.ds(sid * seg, seg)],
                        out_h.at[pl.ds(cid * NSPAD + sid * seg, seg)])
        pltpu.sync_copy(ddeg_sh.at[pl.ds(sid * seg, seg)],
                        out_h.at[pl.ds(2 * NSPAD + cid * NSPAD + sid * seg, seg)])

    return k(srcA, dstA)


def _sc_coeff(srcB, dstB, ew2d, norms_flat):
    rows_per_tec = ERP // 32
    n_chunks = rows_per_tec // RCH

    @functools.partial(
        pl.kernel,
        out_type=jax.ShapeDtypeStruct((ERP, CW), _f32),
        mesh=_mesh(),
        compiler_params=pltpu.CompilerParams(needs_layout_passes=False),
        scratch_types=[
            pltpu.VMEM((RCH, CW), jnp.int32),
            pltpu.VMEM((RCH, CW), jnp.int32),
            pltpu.VMEM((RCH, CW), _f32),
            pltpu.VMEM((RCH, CW), _f32),
            pltpu.VMEM((NSPAD,), _f32),
            pltpu.VMEM((NSPAD,), _f32),
        ],
    )
    def k(s_h, d_h, ew_h, norms_h, c_out_h, sidx_v, didx_v, ew_v, c_v, ns_v, nd_v):
        cid = lax.axis_index("c")
        sid = lax.axis_index("s")
        pltpu.sync_copy(norms_h.at[pl.ds(0, NSPAD)], ns_v)
        pltpu.sync_copy(norms_h.at[pl.ds(NSPAD, NSPAD)], nd_v)

        base = (cid * 16 + sid) * rows_per_tec

        def chunk(t, _):
            r0 = base + t * RCH
            pltpu.sync_copy(s_h.at[pl.ds(r0, RCH)], sidx_v)
            pltpu.sync_copy(d_h.at[pl.ds(r0, RCH)], didx_v)
            pltpu.sync_copy(ew_h.at[pl.ds(r0, RCH)], ew_v)
            for j in range(RCH):
                def g16(g, _):
                    si = sidx_v[j, pl.ds(g * 16, 16)]
                    di = didx_v[j, pl.ds(g * 16, 16)]
                    a = plsc.load_gather(ns_v, [si])
                    b = plsc.load_gather(nd_v, [di])
                    c_v[j, pl.ds(g * 16, 16)] = ew_v[j, pl.ds(g * 16, 16)] * a * b
                    return 0

                lax.fori_loop(0, CW // 16, g16, 0)
            pltpu.sync_copy(c_v, c_out_h.at[pl.ds(r0, RCH)])
            return 0

        lax.fori_loop(0, n_chunks, chunk, 0)

    return k(srcB, dstB, ew2d, norms_flat)


def _sc_spmm(table, srcB, dstB, c2d, *, split_edges):
    D = 128
    rows_per_tec = (ERP // 32) if split_edges else (ERP // 16)
    n_chunks = rows_per_tec // RCH

    @functools.partial(
        pl.kernel,
        out_type=jax.ShapeDtypeStruct((2, NPAD, D), _f32),
        mesh=_mesh(),
        compiler_params=pltpu.CompilerParams(needs_layout_passes=False,
                                             use_tc_tiling_on_sc=False),
        scratch_types=[
            pltpu.VMEM((2, RCH, CW), jnp.int32),
            pltpu.VMEM((2, RCH, CW), jnp.int32),
            pltpu.VMEM((2 * RCH * CW,), _f32),
            pltpu.VMEM((4, CW, D // 2), jnp.int32),
            pltpu.VMEM((2, CW, D), _f32),
            pltpu.VMEM((ZR, D), _f32),
            pltpu.VMEM_SHARED((NPAD, D), _f32),
            pltpu.SemaphoreType.DMA,
            pltpu.SemaphoreType.DMA,
            pltpu.SemaphoreType.DMA,
            pltpu.SemaphoreType.DMA,
            pltpu.SemaphoreType.DMA,
            pltpu.SemaphoreType.DMA,
            pltpu.SemaphoreType.DMA,
        ],
    )
    def k(table_h, s_h, d_h, c_h, out_h, sidx_v, didx_v, c_v, rows_v, frows_v,
          z_v, agg_sh, gs0, gs1, gs2, gs3, ss0, ss1, isem):
        gsems = (gs0, gs1, gs2, gs3)
        ssems = (ss0, ss1)
        cid = lax.axis_index("c")
        sid = lax.axis_index("s")

        def zb(r, _):
            for g in range(D // 16):
                z_v[r, pl.ds(g * 16, 16)] = jnp.zeros((16,), _f32)
            return 0

        lax.fori_loop(0, ZR, zb, 0)

        def zc(t, _):
            pltpu.sync_copy(z_v, agg_sh.at[pl.ds(sid * RPT + t * ZR, ZR)])
            return 0

        lax.fori_loop(0, RPT // ZR, zc, 0)
        plsc.subcore_barrier()

        if split_edges:
            base = cid * (ERP // 2) + sid * rows_per_tec
        else:
            base = sid * rows_per_tec
        off = cid * NPAD

        def idx_descs(slot, r0):
            return (
                pltpu.make_async_copy(s_h.at[pl.ds(r0, RCH)], sidx_v.at[slot], isem),
                pltpu.make_async_copy(d_h.at[pl.ds(r0, RCH)], didx_v.at[slot], isem),
                pltpu.make_async_copy(c_h.at[pl.ds(r0 * CW, RCH * CW)],
                                      c_v.at[pl.ds(slot * RCH * CW, RCH * CW)],
                                      isem),
            )

        def addoff(slot):
            def body(j, _):
                for g in range(CW // 16):
                    sidx_v[slot, j, pl.ds(g * 16, 16)] = (
                        sidx_v[slot, j, pl.ds(g * 16, 16)] + off)
                return 0

            lax.fori_loop(0, RCH, body, 0)

        def fire_gather(slot, j, b):
            pltpu.make_async_copy(
                table_h.at[sidx_v.at[slot, j]], rows_v.at[b], gsems[b]).start()

        def gather_wait(b):
            pltpu.make_async_copy(
                table_h.at[sidx_v.at[0, 0]], rows_v.at[b], gsems[b]).wait()

        def scatter_wait(fb):
            pltpu.make_async_copy(
                frows_v.at[fb], agg_sh.at[didx_v.at[0, 0]], ssems[fb]).wait()

        for d in idx_descs(0, base):
            d.start()
        for d in idx_descs(0, base):
            d.wait()
        addoff(0)
        fire_gather(0, 0, 0)
        fire_gather(0, 1, 1)

        def chunk(t, _):
            slot = lax.rem(t, 2)
            for j in range(RCH):
                b = j % 4
                b2 = (j + 2) % 4
                fb = j % 2
                if j >= 2:
                    scatter_wait(fb)
                else:
                    @pl.when(t > 0)
                    def _():
                        scatter_wait(fb)

                if j == 2:
                    @pl.when(t + 1 < n_chunks)
                    def _():
                        for d in idx_descs(1 - slot, base + (t + 1) * RCH):
                            d.start()

                if j < RCH - 2:
                    fire_gather(slot, j + 2, b2)
                elif j == RCH - 2:
                    @pl.when(t + 1 < n_chunks)
                    def _():
                        for d in idx_descs(1 - slot, base + (t + 1) * RCH):
                            d.wait()
                        addoff(1 - slot)
                        fire_gather(1 - slot, 0, b2)
                else:
                    @pl.when(t + 1 < n_chunks)
                    def _():
                        fire_gather(1 - slot, 1, b2)

                gather_wait(b)
                cbase = (slot * RCH + j) * CW

                def scale(i4, _):
                    for u in range(2):
                        i = i4 * 2 + u
                        ci = jnp.full((16,), cbase + i, jnp.int32)
                        cs = plsc.load_gather(c_v, [ci])
                        for g_ in range(D // 32):
                            vb = plsc.bitcast(
                                rows_v[b, i, pl.ds(g_ * 16, 16)], jnp.bfloat16)
                            lo, hi = plsc.unpack(
                                vb, format=plsc.PackFormat.INTERLEAVED)
                            frows_v[fb, i, pl.ds(g_ * 32, 16)] = lo * cs
                            frows_v[fb, i, pl.ds(g_ * 32 + 16, 16)] = hi * cs
                    return 0

                lax.fori_loop(0, CW // 2, scale, 0)
                pltpu.make_async_copy(
                    frows_v.at[fb], agg_sh.at[didx_v.at[slot, j]],
                    ssems[fb]).start(add=True)
            return 0

        lax.fori_loop(0, n_chunks, chunk, 0)
        scatter_wait(0)
        scatter_wait(1)
        plsc.subcore_barrier()

        def co(t, _):
            r0 = sid * RPT + t * ZR
            pltpu.sync_copy(agg_sh.at[pl.ds(r0, ZR)], out_h.at[cid, pl.ds(r0, ZR)])
            return 0

        lax.fori_loop(0, RPT // ZR, co, 0)

    return k(table, srcB, dstB, c2d)


def _sc_pred(srcB, dstB, p12):
    rows_per_tec = ERP // 32
    n_chunks = rows_per_tec // RCH

    @functools.partial(
        pl.kernel,
        out_type=jax.ShapeDtypeStruct((ERP, CW), _f32),
        mesh=_mesh(),
        compiler_params=pltpu.CompilerParams(needs_layout_passes=False),
        scratch_types=[
            pltpu.VMEM((RCH, CW), jnp.int32),
            pltpu.VMEM((RCH, CW), jnp.int32),
            pltpu.VMEM((RCH, CW), _f32),
            pltpu.VMEM((N,), _f32),
            pltpu.VMEM((N,), _f32),
        ],
    )
    def k(s_h, d_h, p_h, out_h, sidx_v, didx_v, sc_v, p1_v, p2_v):
        cid = lax.axis_index("c")
        sid = lax.axis_index("s")
        pltpu.sync_copy(p_h.at[pl.ds(0, N)], p1_v)
        pltpu.sync_copy(p_h.at[pl.ds(N, N)], p2_v)

        base = (cid * 16 + sid) * rows_per_tec

        def chunk(t, _):
            r0 = base + t * RCH
            pltpu.sync_copy(s_h.at[pl.ds(r0, RCH)], sidx_v)
            pltpu.sync_copy(d_h.at[pl.ds(r0, RCH)], didx_v)
            for j in range(RCH):
                def g16(g, _):
                    si = sidx_v[j, pl.ds(g * 16, 16)]
                    di = didx_v[j, pl.ds(g * 16, 16)]
                    a = plsc.load_gather(p1_v, [si])
                    b = plsc.load_gather(p2_v, [di])
                    sc_v[j, pl.ds(g * 16, 16)] = a + b
                    return 0

                lax.fori_loop(0, CW // 16, g16, 0)
            pltpu.sync_copy(sc_v, out_h.at[pl.ds(r0, RCH)])
            return 0

        lax.fori_loop(0, n_chunks, chunk, 0)

    return k(srcB, dstB, p12)


def _tc_ew(e, w1, b1r, w2, b2r):
    BE = 8000

    def body(e_ref, w1_ref, b1_ref, w2_ref, b2_ref, out_ref):
        wc = jnp.dot(w1_ref[...], w2_ref[...], preferred_element_type=_f32)
        bc = jnp.dot(b1_ref[...], w2_ref[...], preferred_element_type=_f32)
        out_ref[...] = (
            jnp.dot(e_ref[...], wc, preferred_element_type=_f32)
            + bc + b2_ref[...])

    return pl.pallas_call(
        body,
        grid=(E // BE,),
        in_specs=[
            pl.BlockSpec((BE, 16), lambda i: (i, 0)),
            pl.BlockSpec((16, 8), lambda i: (0, 0)),
            pl.BlockSpec((1, 8), lambda i: (0, 0)),
            pl.BlockSpec((8, 1), lambda i: (0, 0)),
            pl.BlockSpec((1, 1), lambda i: (0, 0)),
        ],
        out_specs=pl.BlockSpec((BE, 1), lambda i: (i, 0)),
        out_shape=jax.ShapeDtypeStruct((E, 1), _f32),
    )(e, w1, b1r, w2, b2r)


def _tc_norms(degs):
    def body(d_ref, out_ref):
        s = jnp.maximum(d_ref[0, :] + d_ref[1, :], 1.0)
        d = jnp.maximum(d_ref[2, :] + d_ref[3, :], 1.0)
        out_ref[0, :] = lax.rsqrt(s)
        out_ref[1, :] = lax.rsqrt(d)

    return pl.pallas_call(
        body,
        out_shape=jax.ShapeDtypeStruct((2, NSPAD), _f32),
    )(degs.reshape(4, NSPAD))


def _tc_mm1(agg1, w1, b1r):
    BN = RPT

    def body(a_ref, w_ref, b_ref, out_ref):
        acc = a_ref[0] + a_ref[1]
        out_ref[...] = jax.nn.relu(
            jnp.dot(acc, w_ref[...], preferred_element_type=_f32)
            + b_ref[...]).astype(jnp.bfloat16)

    nb = NPAD // BN
    return pl.pallas_call(
        body,
        grid=(nb, 2),
        in_specs=[
            pl.BlockSpec((2, BN, 128), lambda i, j: (0, i, 0)),
            pl.BlockSpec((128, 128), lambda i, j: (0, j)),
            pl.BlockSpec((1, 128), lambda i, j: (0, j)),
        ],
        out_specs=pl.BlockSpec((BN, 128), lambda i, j: (j * nb + i, 0)),
        out_shape=jax.ShapeDtypeStruct((2 * NPAD, 128), jnp.bfloat16),
    )(agg1, w1, b1r)


def _tc_mm2(agg2, w2t, w2b, b2r, wcat, bcat):
    BN = RPT

    def body(a_ref, wt_ref, wb_ref, b_ref, wc_ref, bc_ref, out_ref):
        h2 = jax.nn.relu(
            jnp.dot(a_ref[0], wt_ref[...], preferred_element_type=_f32)
            + jnp.dot(a_ref[1], wb_ref[...], preferred_element_type=_f32)
            + b_ref[...])
        out_ref[...] = (
            jnp.dot(h2, wc_ref[...], preferred_element_type=_f32) + bc_ref[...])

    nb = NPAD // BN
    return pl.pallas_call(
        body,
        grid=(nb,),
        in_specs=[
            pl.BlockSpec((2, BN, 128), lambda i: (0, i, 0)),
            pl.BlockSpec((128, H), lambda i: (0, 0)),
            pl.BlockSpec((128, H), lambda i: (0, 0)),
            pl.BlockSpec((1, H), lambda i: (0, 0)),
            pl.BlockSpec((H, 128), lambda i: (0, 0)),
            pl.BlockSpec((1, 128), lambda i: (0, 0)),
        ],
        out_specs=pl.BlockSpec((BN, 128), lambda i: (i, 0)),
        out_shape=jax.ShapeDtypeStruct((NPAD, 128), _f32),
    )(agg2, w2t, w2b, b2r, wcat, bcat)


def _pad2d(v, fill):
    pad = jnp.full((ERP * CW - E,), fill, v.dtype)
    return jnp.concatenate([v, pad]).reshape(ERP, CW)


def _ilv_perm(n):
    import numpy as _np
    p = _np.empty(n, _np.int32)
    for b0 in range(0, n, 32):
        for i in range(16):
            p[b0 + 2 * i] = b0 + i
            p[b0 + 2 * i + 1] = b0 + 16 + i
    return p


def kernel(x, edge_index, e, W_lin1, b_lin1, W_lin2, b_lin2,
           W_conv1, b_conv1, W_conv2, b_conv2, W_pred, b_pred):
    src = edge_index[0]
    dst = edge_index[1]
    srcA = _pad2d(src, PADIDX)
    dstA = _pad2d(dst, PADIDX)
    srcB = _pad2d(src, 0)
    dstB = _pad2d(dst, 0)

    degs = _sc_degrees(srcA, dstA)
    norms = _tc_norms(degs).reshape(2 * NSPAD)

    ew = _tc_ew(e, W_lin1, b_lin1.reshape(1, 8), W_lin2, b_lin2.reshape(1, 1))
    c2d = _sc_coeff(srcB, dstB, _pad2d(ew.reshape(E), 0.0), norms)

    perm128 = _ilv_perm(128)
    perm256 = _ilv_perm(256)

    def _pack_i32(a):
        return lax.bitcast_convert_type(
            a.reshape(a.shape[0], 64, 2), jnp.int32)

    xp = jnp.concatenate([x[:, perm128].astype(jnp.bfloat16),
                          jnp.zeros((NPAD - N, 128), jnp.bfloat16)])
    xp2 = jnp.concatenate([xp, xp])
    cflat = c2d.reshape(ERP * CW)
    agg1 = _sc_spmm(_pack_i32(xp2), srcB, dstB, cflat, split_edges=True)
    h1cat = _tc_mm1(agg1, W_conv1[:, perm256], b_conv1[perm256].reshape(1, H))
    agg2 = _sc_spmm(_pack_i32(h1cat), srcB, dstB, cflat, split_edges=False)

    wcat = jnp.zeros((H, 128), _f32)
    wcat = wcat.at[:, 0].set(W_pred[:H, 0]).at[:, 1].set(W_pred[H:, 0])
    bcat = jnp.zeros((1, 128), _f32).at[0, 0].set(b_pred[0])
    P = _tc_mm2(agg2, W_conv2[:128], W_conv2[128:], b_conv2.reshape(1, H),
                wcat, bcat)

    p12 = jnp.concatenate([P[:N, 0], P[:N, 1]])
    score2d = _sc_pred(srcB, dstB, p12)
    return score2d.reshape(ERP * CW)[:E].reshape(E, 1)
